# Initial kernel scaffold; baseline (speedup 1.0000x reference)
import jax
import jax.numpy as jnp
from jax.experimental import pallas as pl


def kernel(weights, params):
    raise NotImplementedError("write your pallas kernel here")


# speedup vs baseline: 8879.3887x; 8879.3887x over previous
import jax
import jax.numpy as jnp
from jax.experimental import pallas as pl
from jax.experimental.pallas import tpu as pltpu

_N = 2048
_BI = 128
_CUTOFF = 5.0


def _pow20(x):
    x2 = x * x
    x4 = x2 * x2
    x8 = x4 * x4
    x16 = x8 * x8
    return x16 * x4


def _eam_block_kernel(w_i_ref, wt_ref, p_i_ref, pt_ref, out_ref):
    i0 = pl.program_id(0) * _BI

    xi = w_i_ref[:, 0:1]
    yi = w_i_ref[:, 1:2]
    zi = w_i_ref[:, 2:3]
    xj = wt_ref[0:1, :]
    yj = wt_ref[1:2, :]
    zj = wt_ref[2:3, :]

    dx = xi - xj
    dy = yi - yj
    dz = zi - zj
    r = jnp.sqrt(dx * dx + dy * dy + dz * dz)

    re_i = p_i_ref[:, 0:1]
    fe_i = p_i_ref[:, 1:2]
    al_i = p_i_ref[:, 4:5]
    be_i = p_i_ref[:, 5:6]
    a_i = p_i_ref[:, 6:7]
    b_i = p_i_ref[:, 7:8]
    ka_i = p_i_ref[:, 8:9]
    la_i = p_i_ref[:, 9:10]

    re_j = pt_ref[0:1, :]
    fe_j = pt_ref[1:2, :]
    al_j = pt_ref[4:5, :]
    be_j = pt_ref[5:6, :]
    a_j = pt_ref[6:7, :]
    b_j = pt_ref[7:8, :]
    ka_j = pt_ref[8:9, :]
    la_j = pt_ref[9:10, :]

    u_i = r / re_i
    um1_i = u_i - 1.0
    eb_i = jnp.exp(-be_i * um1_i)
    ea_i = jnp.exp(-al_i * um1_i)
    dlam_i = 1.0 + _pow20(u_i - la_i)
    dkap_i = 1.0 + _pow20(u_i - ka_i)
    fr_i = fe_i * eb_i / dlam_i
    phir_i = a_i * ea_i / dkap_i - b_i * eb_i / dlam_i

    u_j = r / re_j
    um1_j = u_j - 1.0
    eb_j = jnp.exp(-be_j * um1_j)
    ea_j = jnp.exp(-al_j * um1_j)
    dlam_j = 1.0 + _pow20(u_j - la_j)
    dkap_j = 1.0 + _pow20(u_j - ka_j)
    fr_j = fe_j * eb_j / dlam_j
    phir_j = a_j * ea_j / dkap_j - b_j * eb_j / dlam_j

    phi01 = (fr_j / fr_i) * phir_i + (fr_i / fr_j) * phir_j

    rows = i0 + jax.lax.broadcasted_iota(jnp.int32, (_BI, _N), 0)
    cols = jax.lax.broadcasted_iota(jnp.int32, (_BI, _N), 1)
    offdiag = rows != cols

    pair_mask = jnp.logical_and(r <= _CUTOFF, offdiag)
    pair_part = 0.25 * jnp.sum(jnp.where(pair_mask, phi01, 0.0),
                               axis=(0, 1), keepdims=True)

    rho = jnp.sum(jnp.where(offdiag, fr_j, 0.0), axis=1, keepdims=True)

    f_n0 = p_i_ref[:, 10:11]
    f_n1 = p_i_ref[:, 11:12]
    f_n2 = p_i_ref[:, 12:13]
    f_n3 = p_i_ref[:, 13:14]
    f_0 = p_i_ref[:, 14:15]
    f_1 = p_i_ref[:, 15:16]
    f_2 = p_i_ref[:, 16:17]
    f_3 = p_i_ref[:, 17:18]
    f_e = p_i_ref[:, 19:20]
    rho_n = p_i_ref[:, 20:21]
    rho_e = p_i_ref[:, 2:3]
    rho_0 = p_i_ref[:, 21:22]
    rho_s = p_i_ref[:, 3:4]
    eta = p_i_ref[:, 18:19]

    t_n = rho / rho_n - 1.0
    b1 = f_n0 + (f_n1 + (f_n2 + f_n3 * t_n) * t_n) * t_n
    t_e = rho / rho_e - 1.0
    b2 = f_0 + (f_1 + (f_2 + f_3 * t_e) * t_e) * t_e
    ratio = rho / rho_s
    lpw = eta * jnp.log(ratio)
    pw = jnp.exp(lpw)
    b3 = f_e * (1.0 - lpw) * pw
    f_val = jnp.where(rho < rho_n, b1, jnp.where(rho < rho_0, b2, b3))

    total = pair_part + jnp.sum(f_val, axis=(0, 1), keepdims=True)
    out_ref[0] = total


def kernel(weights, params):
    n = weights.shape[0]
    k = n // _BI
    wt = weights.T
    pt = params.T
    partials = pl.pallas_call(
        _eam_block_kernel,
        grid=(k,),
        in_specs=[
            pl.BlockSpec((_BI, 3), lambda i: (i, 0)),
            pl.BlockSpec((3, n), lambda i: (0, 0)),
            pl.BlockSpec((_BI, 22), lambda i: (i, 0)),
            pl.BlockSpec((22, n), lambda i: (0, 0)),
        ],
        out_specs=pl.BlockSpec((1, 1, 1), lambda i: (i, 0, 0)),
        out_shape=jax.ShapeDtypeStruct((k, 1, 1), jnp.float32),
    )(weights, wt, params, pt)
    return jnp.sum(partials)
